# baseline (device time: 41334 ns/iter reference)
import jax
import jax.numpy as jnp
from jax import lax
from jax.experimental import pallas as pl
from jax.experimental.pallas import tpu as pltpu

N_DEV = 8
B, S, H, Dh, Dr = 2, 256, 16, 64, 32
D = 1024
DC = 64
BS = B * S
HL = H // N_DEV
CK = HL * Dh
CR = HL * Dr


def _dot(a, b):
    return jnp.dot(a, b, preferred_element_type=jnp.float32)


def kernel(x, Wdkv, Wuk, Wuv, Wq, Wqr, Wkr, Wo):
    def body(x_any, wdkv_ref, wuk_ref, wuv_ref, wkr_ref,
             wq_any, wqr_any, wo_any, out_ref,
             c_mine, w_send, o_mine, c_buf, w_buf, o_buf,
             x_v, wq_v, wqr_v, wo_v,
             c_sems, w_sems, o_sems, in_sems):
        my = lax.axis_index("i")
        bf16 = jnp.bfloat16

        x_copy = pltpu.make_async_copy(x_any, x_v, in_sems.at[0])
        wq_copy = pltpu.make_async_copy(wq_any, wq_v, in_sems.at[1])
        wqr_copy = pltpu.make_async_copy(wqr_any, wqr_v, in_sems.at[2])
        wo_copy = pltpu.make_async_copy(wo_any, wo_v, in_sems.at[3])
        x_copy.start()
        wq_copy.start()
        wqr_copy.start()
        wo_copy.start()

        barrier = pltpu.get_barrier_semaphore()
        for k in range(1, N_DEV):
            pl.semaphore_signal(barrier, inc=1, device_id=((my + k) % N_DEV,),
                                device_id_type=pl.DeviceIdType.MESH)

        x_copy.wait()
        for b in range(B):
            c_mine[b * S:(b + 1) * S, :] = _dot(
                x_v[b], wdkv_ref[...]).astype(bf16)
        wuk = wuk_ref[...]
        wuv = wuv_ref[...]
        for d in range(N_DEV):
            w_send[d, :, :CK] = wuk[:, d * CK:(d + 1) * CK].astype(bf16)
            w_send[d, :, CK:] = wuv[:, d * CK:(d + 1) * CK].astype(bf16)

        pl.semaphore_wait(barrier, N_DEV - 1)

        sends = []
        for k in range(1, N_DEV):
            peer = (my + k) % N_DEV
            slot = N_DEV - 1 - k
            c_rdma = pltpu.make_async_remote_copy(
                src_ref=c_mine, dst_ref=c_buf.at[slot],
                send_sem=c_sems.at[0, k - 1], recv_sem=c_sems.at[1, slot],
                device_id=(peer,), device_id_type=pl.DeviceIdType.MESH)
            w_rdma = pltpu.make_async_remote_copy(
                src_ref=w_send.at[peer], dst_ref=w_buf.at[slot],
                send_sem=w_sems.at[0, k - 1], recv_sem=w_sems.at[1, slot],
                device_id=(peer,), device_id_type=pl.DeviceIdType.MESH)
            c_rdma.start()
            w_rdma.start()
            sends.append((c_rdma, w_rdma))

        wq_copy.wait()
        wqr_copy.wait()
        wq_my = wq_v[:, pl.ds(my * CK, CK)]
        wqr_win = wqr_v[:, pl.ds((my // 2) * (2 * CR), 2 * CR)]
        is_even = (my % 2) == 0
        Qs, Qrs, Krs = [], [], []
        for b in range(B):
            xb = x_v[b]
            Qs.append(_dot(xb, wq_my))
            qr_pair = _dot(xb, wqr_win)
            Qrs.append(jnp.where(is_even, qr_pair[:, :CR], qr_pair[:, CR:]))
            Krs.append(_dot(xb, wkr_ref[...]))

        w_own = w_send[pl.ds(my, 1)][0]
        c_own = c_mine[...]
        K = _dot(c_own, w_own[:, :CK])
        V = _dot(c_own, w_own[:, CK:])
        for j in range(N_DEV - 1):
            recv_c = pltpu.make_async_remote_copy(
                src_ref=c_buf.at[j], dst_ref=c_buf.at[j],
                send_sem=c_sems.at[0, j], recv_sem=c_sems.at[1, j],
                device_id=(my,), device_id_type=pl.DeviceIdType.MESH)
            recv_w = pltpu.make_async_remote_copy(
                src_ref=w_buf.at[j], dst_ref=w_buf.at[j],
                send_sem=w_sems.at[0, j], recv_sem=w_sems.at[1, j],
                device_id=(my,), device_id_type=pl.DeviceIdType.MESH)
            recv_c.wait_recv()
            recv_w.wait_recv()
            K = K + _dot(c_buf[j], w_buf[j, :, :CK])
            V = V + _dot(c_buf[j], w_buf[j, :, CK:])

        scale = (Dh + Dr) ** -0.5
        o_sends = []
        for b in range(B):
            Qb, Qrb, Krb = Qs[b], Qrs[b], Krs[b]
            Kb = K[b * S:(b + 1) * S, :]
            Vb = V[b * S:(b + 1) * S, :]
            for h in range(HL):
                Qh = Qb[:, h * Dh:(h + 1) * Dh]
                Kh = Kb[:, h * Dh:(h + 1) * Dh]
                Qrh = Qrb[:, h * Dr:(h + 1) * Dr]
                s = (_dot(Qh, Kh.T) + _dot(Qrh, Krb.T)) * scale
                m = jnp.max(s, axis=-1, keepdims=True)
                e = jnp.exp(s - m)
                p = e / jnp.sum(e, axis=-1, keepdims=True)
                o_mine[b * S:(b + 1) * S, h * Dh:(h + 1) * Dh] = _dot(
                    p, Vb[:, h * Dh:(h + 1) * Dh]).astype(bf16)
            for k in range(1, N_DEV):
                peer = (my + k) % N_DEV
                slot = N_DEV - 1 - k
                o_rdma = pltpu.make_async_remote_copy(
                    src_ref=o_mine.at[b * S:(b + 1) * S, :],
                    dst_ref=o_buf.at[slot, b * S:(b + 1) * S, :],
                    send_sem=o_sems.at[b, 0, k - 1],
                    recv_sem=o_sems.at[b, 1, slot],
                    device_id=(peer,), device_id_type=pl.DeviceIdType.MESH)
                o_rdma.start()
                o_sends.append(o_rdma)

        wo_copy.wait()
        wo_my = wo_v[pl.ds(my * CK, CK), :]
        out = _dot(o_mine[...], wo_my.astype(bf16))
        for j in range(N_DEV - 1):
            for b in range(B):
                recv_o = pltpu.make_async_remote_copy(
                    src_ref=o_buf.at[j, b * S:(b + 1) * S, :],
                    dst_ref=o_buf.at[j, b * S:(b + 1) * S, :],
                    send_sem=o_sems.at[b, 0, j], recv_sem=o_sems.at[b, 1, j],
                    device_id=(my,), device_id_type=pl.DeviceIdType.MESH)
                recv_o.wait_recv()
            src = (my + j + 1) % N_DEV
            wo_s = wo_v[pl.ds(src * CK, CK), :]
            out = out + _dot(o_buf[j], wo_s.astype(bf16))
        for b in range(B):
            out_ref[b, :, :] = out[b * S:(b + 1) * S, :]

        for c_rdma, w_rdma in sends:
            c_rdma.wait_send()
            w_rdma.wait_send()
        for o_rdma in o_sends:
            o_rdma.wait_send()

    f32 = jnp.float32
    bf16 = jnp.bfloat16
    return pl.pallas_call(
        body,
        out_shape=jax.ShapeDtypeStruct((B, S, D), f32),
        in_specs=(
            [pl.BlockSpec(memory_space=pl.ANY)]
            + [pl.BlockSpec(memory_space=pltpu.VMEM)] * 4
            + [pl.BlockSpec(memory_space=pl.ANY)] * 3
        ),
        out_specs=pl.BlockSpec(memory_space=pltpu.VMEM),
        scratch_shapes=[
            pltpu.VMEM((BS, DC), bf16),
            pltpu.VMEM((N_DEV, DC, 2 * CK), bf16),
            pltpu.VMEM((BS, CK), bf16),
            pltpu.VMEM((N_DEV - 1, BS, DC), bf16),
            pltpu.VMEM((N_DEV - 1, DC, 2 * CK), bf16),
            pltpu.VMEM((N_DEV - 1, BS, CK), bf16),
            pltpu.VMEM((B, S, D), f32),
            pltpu.VMEM((D, D), f32),
            pltpu.VMEM((D, H * Dr), f32),
            pltpu.VMEM((D, D), f32),
            pltpu.SemaphoreType.DMA((2, N_DEV - 1)),
            pltpu.SemaphoreType.DMA((2, N_DEV - 1)),
            pltpu.SemaphoreType.DMA((B, 2, N_DEV - 1)),
            pltpu.SemaphoreType.DMA((4,)),
        ],
        compiler_params=pltpu.CompilerParams(collective_id=0),
    )(x, Wdkv, Wuk, Wuv, Wkr, Wq, Wqr, Wo)


# device time: 33423 ns/iter; 1.2367x vs baseline; 1.2367x over previous
import jax
import jax.numpy as jnp
from jax import lax
from jax.experimental import pallas as pl
from jax.experimental.pallas import tpu as pltpu

N_DEV = 8
B, S, H, Dh, Dr = 2, 256, 16, 64, 32
D = 1024
DC = 64
BS = B * S
HL = H // N_DEV
CK = HL * Dh
CR = HL * Dr


def _dot(a, b):
    return jnp.dot(a, b, preferred_element_type=jnp.float32)


def kernel(x, Wdkv, Wuk, Wuv, Wq, Wqr, Wkr, Wo):
    my_out = lax.axis_index("i")
    wq_my = lax.dynamic_slice(Wq, (0, my_out * CK), (D, CK))
    wqr_my = lax.dynamic_slice(Wqr, (0, my_out * CR), (D, CR))

    def body(x_ref, wdkv_ref, wuk_ref, wuv_ref, wkr_ref,
             wq_my_ref, wqr_my_ref, wo_any, out_ref,
             c_mine, w_send, o_mine, c_buf, w_buf, o_buf,
             wo_v,
             c_sems, w_sems, o_sems, wo_sem):
        my = lax.axis_index("i")
        bf16 = jnp.bfloat16

        wo_copy = pltpu.make_async_copy(wo_any, wo_v, wo_sem)
        wo_copy.start()

        barrier = pltpu.get_barrier_semaphore()
        for k in range(1, N_DEV):
            pl.semaphore_signal(barrier, inc=1, device_id=((my + k) % N_DEV,),
                                device_id_type=pl.DeviceIdType.MESH)

        for b in range(B):
            c_mine[b * S:(b + 1) * S, :] = _dot(
                x_ref[b], wdkv_ref[...]).astype(bf16)
        wuk = wuk_ref[...]
        wuv = wuv_ref[...]
        for d in range(N_DEV):
            w_send[d, :, :CK] = wuk[:, d * CK:(d + 1) * CK].astype(bf16)
            w_send[d, :, CK:] = wuv[:, d * CK:(d + 1) * CK].astype(bf16)

        pl.semaphore_wait(barrier, N_DEV - 1)

        sends = []
        for k in range(1, N_DEV):
            peer = (my + k) % N_DEV
            slot = N_DEV - 1 - k
            c_rdma = pltpu.make_async_remote_copy(
                src_ref=c_mine, dst_ref=c_buf.at[slot],
                send_sem=c_sems.at[0, k - 1], recv_sem=c_sems.at[1, slot],
                device_id=(peer,), device_id_type=pl.DeviceIdType.MESH)
            w_rdma = pltpu.make_async_remote_copy(
                src_ref=w_send.at[peer], dst_ref=w_buf.at[slot],
                send_sem=w_sems.at[0, k - 1], recv_sem=w_sems.at[1, slot],
                device_id=(peer,), device_id_type=pl.DeviceIdType.MESH)
            c_rdma.start()
            w_rdma.start()
            sends.append((c_rdma, w_rdma))

        Qs, Qrs, Krs = [], [], []
        for b in range(B):
            xb = x_ref[b]
            Qs.append(_dot(xb, wq_my_ref[...]))
            Qrs.append(_dot(xb, wqr_my_ref[...]))
            Krs.append(_dot(xb, wkr_ref[...]))

        w_own = w_send[pl.ds(my, 1)][0]
        c_own = c_mine[...]
        K = _dot(c_own, w_own[:, :CK])
        V = _dot(c_own, w_own[:, CK:])
        for j in range(N_DEV - 1):
            recv_c = pltpu.make_async_remote_copy(
                src_ref=c_buf.at[j], dst_ref=c_buf.at[j],
                send_sem=c_sems.at[0, j], recv_sem=c_sems.at[1, j],
                device_id=(my,), device_id_type=pl.DeviceIdType.MESH)
            recv_w = pltpu.make_async_remote_copy(
                src_ref=w_buf.at[j], dst_ref=w_buf.at[j],
                send_sem=w_sems.at[0, j], recv_sem=w_sems.at[1, j],
                device_id=(my,), device_id_type=pl.DeviceIdType.MESH)
            recv_c.wait_recv()
            recv_w.wait_recv()
            K = K + _dot(c_buf[j], w_buf[j, :, :CK])
            V = V + _dot(c_buf[j], w_buf[j, :, CK:])

        scale = (Dh + Dr) ** -0.5
        o_sends = []
        for b in range(B):
            Qb, Qrb, Krb = Qs[b], Qrs[b], Krs[b]
            Kb = K[b * S:(b + 1) * S, :]
            Vb = V[b * S:(b + 1) * S, :]
            for h in range(HL):
                Qh = Qb[:, h * Dh:(h + 1) * Dh]
                Kh = Kb[:, h * Dh:(h + 1) * Dh]
                Qrh = Qrb[:, h * Dr:(h + 1) * Dr]
                s = (_dot(Qh, Kh.T) + _dot(Qrh, Krb.T)) * scale
                m = jnp.max(s, axis=-1, keepdims=True)
                e = jnp.exp(s - m)
                p = e / jnp.sum(e, axis=-1, keepdims=True)
                o_mine[b * S:(b + 1) * S, h * Dh:(h + 1) * Dh] = _dot(
                    p, Vb[:, h * Dh:(h + 1) * Dh]).astype(bf16)
            for k in range(1, N_DEV):
                peer = (my + k) % N_DEV
                slot = N_DEV - 1 - k
                o_rdma = pltpu.make_async_remote_copy(
                    src_ref=o_mine.at[b * S:(b + 1) * S, :],
                    dst_ref=o_buf.at[slot, b * S:(b + 1) * S, :],
                    send_sem=o_sems.at[b, 0, k - 1],
                    recv_sem=o_sems.at[b, 1, slot],
                    device_id=(peer,), device_id_type=pl.DeviceIdType.MESH)
                o_rdma.start()
                o_sends.append(o_rdma)

        wo_copy.wait()
        wo_my = wo_v[pl.ds(my * CK, CK), :]
        out = _dot(o_mine[...], wo_my.astype(bf16))
        for j in range(N_DEV - 1):
            for b in range(B):
                recv_o = pltpu.make_async_remote_copy(
                    src_ref=o_buf.at[j, b * S:(b + 1) * S, :],
                    dst_ref=o_buf.at[j, b * S:(b + 1) * S, :],
                    send_sem=o_sems.at[b, 0, j], recv_sem=o_sems.at[b, 1, j],
                    device_id=(my,), device_id_type=pl.DeviceIdType.MESH)
                recv_o.wait_recv()
            src = (my + j + 1) % N_DEV
            wo_s = wo_v[pl.ds(src * CK, CK), :]
            out = out + _dot(o_buf[j], wo_s.astype(bf16))
        for b in range(B):
            out_ref[b, :, :] = out[b * S:(b + 1) * S, :]

        for c_rdma, w_rdma in sends:
            c_rdma.wait_send()
            w_rdma.wait_send()
        for o_rdma in o_sends:
            o_rdma.wait_send()

    f32 = jnp.float32
    bf16 = jnp.bfloat16
    return pl.pallas_call(
        body,
        out_shape=jax.ShapeDtypeStruct((B, S, D), f32),
        in_specs=(
            [pl.BlockSpec(memory_space=pltpu.VMEM)] * 7
            + [pl.BlockSpec(memory_space=pl.ANY)]
        ),
        out_specs=pl.BlockSpec(memory_space=pltpu.VMEM),
        scratch_shapes=[
            pltpu.VMEM((BS, DC), bf16),
            pltpu.VMEM((N_DEV, DC, 2 * CK), bf16),
            pltpu.VMEM((BS, CK), bf16),
            pltpu.VMEM((N_DEV - 1, BS, DC), bf16),
            pltpu.VMEM((N_DEV - 1, DC, 2 * CK), bf16),
            pltpu.VMEM((N_DEV - 1, BS, CK), bf16),
            pltpu.VMEM((D, D), f32),
            pltpu.SemaphoreType.DMA((2, N_DEV - 1)),
            pltpu.SemaphoreType.DMA((2, N_DEV - 1)),
            pltpu.SemaphoreType.DMA((B, 2, N_DEV - 1)),
            pltpu.SemaphoreType.DMA,
        ],
        compiler_params=pltpu.CompilerParams(collective_id=0),
    )(x, Wdkv, Wuk, Wuv, Wkr, wq_my, wqr_my, Wo)
